# baseline (speedup 1.0000x reference)
import jax
import jax.numpy as jnp
from jax import lax
from jax.experimental import pallas as pl
from jax.experimental.pallas import tpu as pltpu
from jax.experimental.pallas import tpu_sc as plsc

_N_ROWS = 4096
_N_FIELDS = 26
_FIELD_SIZE = 1000
_ROW_WORDS = _N_FIELDS * _FIELD_SIZE
_NUM_CORES = 2
_NUM_SUBCORES = 16
_NUM_WORKERS = _NUM_CORES * _NUM_SUBCORES
_ROWS_PER_W = _N_ROWS // _NUM_WORKERS
_CODES_PER_W = _ROWS_PER_W * _N_FIELDS


def _sc_body(inp_ref, out_ref, buf, codes):
    wid = lax.axis_index("s") * _NUM_CORES + lax.axis_index("c")
    pltpu.sync_copy(inp_ref.at[pl.ds(wid * _CODES_PER_W, _CODES_PER_W)], codes)

    zeros = jnp.zeros((16,), jnp.float32)
    ones = jnp.ones((16,), jnp.float32)
    iota = lax.iota(jnp.int32, 16)
    mask_hi = iota >= 6

    def zero_body(i, carry):
        buf[pl.ds(i * 16, 16)] = zeros
        return carry

    lax.fori_loop(0, _ROW_WORDS // 16, zero_body, 0)

    def row_positions(rl):
        c0 = codes[pl.ds(rl * _N_FIELDS, 16)].astype(jnp.int32)
        c1 = codes[pl.ds(rl * _N_FIELDS + 10, 16)].astype(jnp.int32)
        pos0 = iota * _FIELD_SIZE + c0
        pos1 = (iota + 10) * _FIELD_SIZE + c1
        return pos0, pos1

    def row_body(rl, carry):
        pos0, pos1 = row_positions(rl)
        plsc.store_scatter(buf, [pos0], ones)
        plsc.store_scatter(buf, [pos1], ones, mask=mask_hi)
        row = wid * _ROWS_PER_W + rl
        pltpu.sync_copy(buf, out_ref.at[pl.ds(row * _ROW_WORDS, _ROW_WORDS)])
        plsc.store_scatter(buf, [pos0], zeros)
        plsc.store_scatter(buf, [pos1], zeros, mask=mask_hi)
        return carry

    lax.fori_loop(0, _ROWS_PER_W, row_body, 0)


def kernel(input):
    n = input.shape[0]
    flat_in = input.reshape(-1)
    mesh = plsc.VectorSubcoreMesh(
        core_axis_name="c", subcore_axis_name="s"
    )
    out = pl.kernel(
        _sc_body,
        out_type=jax.ShapeDtypeStruct((n * _ROW_WORDS,), jnp.float32),
        mesh=mesh,
        compiler_params=pltpu.CompilerParams(needs_layout_passes=False),
        scratch_types=[
            pltpu.VMEM((_ROW_WORDS,), jnp.float32),
            pltpu.VMEM((_CODES_PER_W,), jnp.float32),
        ],
    )(flat_in)
    return out.reshape(n, _ROW_WORDS)
